# baseline (speedup 1.0000x reference)
import functools

import jax
import jax.numpy as jnp
from jax import lax
from jax.experimental import pallas as pl
from jax.experimental.pallas import tpu as pltpu
from jax.experimental.pallas import tpu_sc as plsc

_EMB = 64
_NC = 2
_NS = 16
_NW = _NC * _NS

_CHUNK = 640
_SUB = 128
_NSUB = _CHUNK // _SUB

_NBT = 64


def _sc_gather(t_flat, table, n_rows):
    b_per_w = n_rows // _NW
    n_chunks = b_per_w // _CHUNK
    n_pairs = n_chunks // 2

    mesh = plsc.VectorSubcoreMesh(core_axis_name="c", subcore_axis_name="s")

    @functools.partial(
        pl.kernel,
        mesh=mesh,
        out_type=jax.ShapeDtypeStruct((n_rows, _EMB), jnp.float32),
        scratch_types=[
            pltpu.VMEM((b_per_w,), jnp.int32),
            pltpu.VMEM((_CHUNK, _EMB), jnp.float32),
            pltpu.VMEM((_CHUNK, _EMB), jnp.float32),
            pltpu.SemaphoreType.DMA,
            pltpu.SemaphoreType.DMA,
            pltpu.SemaphoreType.DMA,
            pltpu.SemaphoreType.DMA,
        ],
        compiler_params=pltpu.CompilerParams(use_tc_tiling_on_sc=False),
    )
    def k(t_hbm, table_hbm, out_hbm, idx_v, rows0, rows1, gs0, gs1, os0, os1):
        wid = lax.axis_index("s") * _NC + lax.axis_index("c")
        base = wid * b_per_w

        pltpu.sync_copy(t_hbm.at[pl.ds(base, b_per_w)], idx_v)

        def fire_gather(c, rows, sem):
            for j in range(_NSUB):
                pltpu.async_copy(
                    table_hbm.at[idx_v.at[pl.ds(c * _CHUNK + j * _SUB, _SUB)]],
                    rows.at[pl.ds(j * _SUB, _SUB)],
                    sem)

        def wait_gather(rows, sem):
            pltpu.make_async_copy(
                table_hbm.at[idx_v.at[pl.ds(0, _SUB)]],
                rows, sem).wait()

        def fire_wb(c, rows, sem):
            pltpu.async_copy(
                rows, out_hbm.at[pl.ds(base + c * _CHUNK, _CHUNK)], sem)

        def wait_wb(rows, sem):
            pltpu.make_async_copy(
                rows, out_hbm.at[pl.ds(0, _CHUNK)], sem).wait()

        fire_gather(0, rows0, gs0)
        fire_gather(1, rows1, gs1)

        def body(i, carry):
            c0 = 2 * i
            wait_gather(rows0, gs0)
            fire_wb(c0, rows0, os0)
            wait_gather(rows1, gs1)
            fire_wb(c0 + 1, rows1, os1)
            wait_wb(rows0, os0)
            fire_gather(c0 + 2, rows0, gs0)
            wait_wb(rows1, os1)
            fire_gather(c0 + 3, rows1, gs1)
            return carry

        lax.fori_loop(0, n_pairs - 1, body, 0)

        c_last = n_chunks - 2
        wait_gather(rows0, gs0)
        fire_wb(c_last, rows0, os0)
        wait_gather(rows1, gs1)
        fire_wb(c_last + 1, rows1, os1)
        wait_wb(rows0, os0)
        wait_wb(rows1, os1)

    return k(t_flat, table)


def _tc_relayout(pairs3, n_batch, seq):
    half = seq // 2

    def body(in_ref, out_ref):
        for kk in range(half):
            out_ref[:, 2 * kk, :] = in_ref[kk, :, 0:_EMB]
            out_ref[:, 2 * kk + 1, :] = in_ref[kk, :, _EMB:2 * _EMB]

    return pl.pallas_call(
        body,
        grid=(n_batch // _NBT,),
        in_specs=[pl.BlockSpec((half, _NBT, 2 * _EMB), lambda i: (0, i, 0))],
        out_specs=pl.BlockSpec((_NBT, seq, _EMB), lambda i: (i, 0, 0)),
        out_shape=jax.ShapeDtypeStruct((n_batch, seq, _EMB), jnp.float32),
    )(pairs3)


def kernel(t, pos_embeddings):
    b, s = t.shape
    t2 = t.reshape(b, s // 2, 2).transpose(1, 0, 2).reshape(-1)
    flat = _sc_gather(t2, pos_embeddings, b * s)
    pairs3 = flat.reshape(s // 2, b, 2 * _EMB)
    return _tc_relayout(pairs3, b, s)


# speedup vs baseline: 1.7958x; 1.7958x over previous
import functools

import jax
import jax.numpy as jnp
from jax import lax
from jax.experimental import pallas as pl
from jax.experimental.pallas import tpu as pltpu
from jax.experimental.pallas import tpu_sc as plsc

_EMB = 64
_NC = 2
_NS = 16
_NW = _NC * _NS

_CHUNK = 640
_SUB = 128
_NSUB = _CHUNK // _SUB

_NBT = 64


def _sc_gather(t_flat, table, n_rows):
    b_per_w = n_rows // _NW
    n_chunks = b_per_w // _CHUNK
    n_pairs = n_chunks // 2

    mesh = plsc.VectorSubcoreMesh(core_axis_name="c", subcore_axis_name="s")

    @functools.partial(
        pl.kernel,
        mesh=mesh,
        out_type=jax.ShapeDtypeStruct((n_rows, _EMB), jnp.float32),
        scratch_types=[
            pltpu.VMEM((b_per_w,), jnp.int32),
            pltpu.VMEM((_CHUNK, _EMB), jnp.float32),
            pltpu.VMEM((_CHUNK, _EMB), jnp.float32),
            pltpu.SemaphoreType.DMA,
            pltpu.SemaphoreType.DMA,
            pltpu.SemaphoreType.DMA,
            pltpu.SemaphoreType.DMA,
        ],
        compiler_params=pltpu.CompilerParams(use_tc_tiling_on_sc=False),
    )
    def k(t_hbm, table_hbm, out_hbm, idx_v, rows0, rows1, gs0, gs1, os0, os1):
        wid = lax.axis_index("s") * _NC + lax.axis_index("c")
        base = wid * b_per_w

        pltpu.sync_copy(t_hbm.at[pl.ds(base, b_per_w)], idx_v)

        def fire_gather(c, rows, sem):
            for j in range(_NSUB):
                pltpu.async_copy(
                    table_hbm.at[idx_v.at[pl.ds(c * _CHUNK + j * _SUB, _SUB)]],
                    rows.at[pl.ds(j * _SUB, _SUB)],
                    sem)

        def wait_gather(rows, sem):
            pltpu.make_async_copy(
                table_hbm.at[idx_v.at[pl.ds(0, _SUB)]],
                rows, sem).wait()

        def fire_wb(c, rows, sem):
            pltpu.async_copy(
                rows, out_hbm.at[pl.ds(base + c * _CHUNK, _CHUNK)], sem)

        def wait_wb(rows, sem):
            pltpu.make_async_copy(
                rows, out_hbm.at[pl.ds(0, _CHUNK)], sem).wait()

        fire_gather(0, rows0, gs0)
        fire_gather(1, rows1, gs1)

        def body(i, carry):
            c0 = 2 * i
            wait_gather(rows0, gs0)
            fire_wb(c0, rows0, os0)
            wait_gather(rows1, gs1)
            fire_wb(c0 + 1, rows1, os1)
            wait_wb(rows0, os0)
            fire_gather(c0 + 2, rows0, gs0)
            wait_wb(rows1, os1)
            fire_gather(c0 + 3, rows1, gs1)
            return carry

        lax.fori_loop(0, n_pairs - 1, body, 0)

        c_last = n_chunks - 2
        wait_gather(rows0, gs0)
        fire_wb(c_last, rows0, os0)
        wait_gather(rows1, gs1)
        fire_wb(c_last + 1, rows1, os1)
        wait_wb(rows0, os0)
        wait_wb(rows1, os1)

    return k(t_flat, table)


def kernel(t, pos_embeddings):
    b, s = t.shape
    flat = _sc_gather(t.reshape(-1), pos_embeddings, b * s)
    return flat.reshape(b, s, _EMB) * jnp.float32(1.0)
